# baseline (speedup 1.0000x reference)
import functools

import numpy as np
import jax
import jax.numpy as jnp
from jax import lax
from jax.experimental import pallas as pl
from jax.experimental.pallas import tpu as pltpu
from jax.experimental.pallas import tpu_sc as plsc

VOCAB = 63
VOCAB_PAD = 64
D = 16
SEQ = 19
BATCH = 16384
TOKENS = BATCH * SEQ

NC = 2
NS = 16
NW = NC * NS
BBLK = BATCH // 128
UNITS = SEQ * BBLK
UNITS_W = UNITS // NW
PER_W = UNITS_W * 128
ROWS = SEQ * VOCAB_PAD
QUADS_W = UNITS_W // 4


def _pe_np() -> np.ndarray:
    even_i = np.arange(0, D, 2, dtype=np.float32)
    denom = np.power(np.float32(10000.0), even_i / np.float32(D))
    pos = np.arange(SEQ, dtype=np.float32).reshape(SEQ, 1)
    stacked = np.stack([np.sin(pos / denom), np.cos(pos / denom)], axis=-1)
    return stacked.reshape(SEQ, D).astype(np.float32)


_PE = _pe_np()


def _fuse_body(tab_ref, pe_ref, out_ref):
    tab_t = jnp.transpose(tab_ref[...], (1, 0))
    pe_t = jnp.transpose(pe_ref[...], (1, 0))
    out_ref[...] = pe_t[:, :, None] + tab_t[:, None, :]


_fuse = pl.pallas_call(
    _fuse_body,
    out_shape=jax.ShapeDtypeStruct((D, SEQ, VOCAB_PAD), jnp.float32),
)


def _sc_body(idx_hbm, fusedt_hbm, out_hbm, idx_v, tab_v, tile_v, ssem0,
             ssem1):
    wid = lax.axis_index("s") * NC + lax.axis_index("c")

    pltpu.sync_copy(idx_hbm.at[pl.ds(wid * PER_W, PER_W)], idx_v)
    pltpu.sync_copy(fusedt_hbm, tab_v)

    sems = (ssem0, ssem1)

    def _do_quad(q, k):
        u0 = wid * UNITS_W + q * 4
        p = lax.div(u0, BBLK)
        bblk = lax.rem(u0, BBLK)
        base = p * VOCAB_PAD
        dvec = [jnp.full((16,), d * ROWS, jnp.int32) + base for d in range(D)]
        for bq in range(4):
            for g in range(8):
                tok = idx_v[pl.ds((q * 4 + bq) * 128 + g * 16, 16)]
                for d in range(D):
                    tile_v[k, d // 8, bq, d % 8, pl.ds(g * 16, 16)] = (
                        plsc.load_gather(tab_v, [tok + dvec[d]]))
        pltpu.async_copy(tile_v.at[k, 0], out_hbm.at[p, 0, pl.ds(bblk, 4)],
                         sems[k])
        pltpu.async_copy(tile_v.at[k, 1], out_hbm.at[p, 1, pl.ds(bblk, 4)],
                         sems[k])

    def _drain(k):
        for db in range(2):
            pltpu.make_async_copy(out_hbm.at[0, 0, pl.ds(0, 4)],
                                  tile_v.at[k, db], sems[k]).wait()

    def _pair(j, _):
        for k in range(2):
            @pl.when(j >= 1)
            def _wait():
                _drain(k)
            _do_quad(2 * j + k, k)
        return _

    lax.fori_loop(0, QUADS_W // 2, _pair, None)
    _drain(0)
    _do_quad(QUADS_W - 1, 0)
    _drain(0)
    _drain(1)


_sc_call = functools.partial(
    pl.kernel,
    out_type=jax.ShapeDtypeStruct((SEQ, 2, BBLK, 8, 128), jnp.float32),
    mesh=plsc.VectorSubcoreMesh(core_axis_name="c", subcore_axis_name="s"),
    compiler_params=pltpu.CompilerParams(use_tc_tiling_on_sc=False,
                                         needs_layout_passes=False),
    scratch_types=[
        pltpu.VMEM((PER_W,), jnp.int32),
        pltpu.VMEM((D * ROWS,), jnp.float32),
        pltpu.VMEM((2, 2, 4, 8, 128), jnp.float32),
        pltpu.SemaphoreType.DMA,
        pltpu.SemaphoreType.DMA,
    ],
)(_sc_body)


def kernel(batch, embedding_table):
    tab = jnp.pad(embedding_table.astype(jnp.float32),
                  ((0, VOCAB_PAD - VOCAB), (0, 0)))
    fusedt = _fuse(tab, jnp.asarray(_PE))
    fusedt = fusedt.reshape(D * ROWS)
    idx = batch.astype(jnp.int32).T.reshape(TOKENS)
    out5 = _sc_call(idx, fusedt)
    out = out5.transpose(2, 4, 0, 1, 3).reshape(BATCH, SEQ, D)
    return out


# speedup vs baseline: 1.0683x; 1.0683x over previous
import functools

import numpy as np
import jax
import jax.numpy as jnp
from jax import lax
from jax.experimental import pallas as pl
from jax.experimental.pallas import tpu as pltpu
from jax.experimental.pallas import tpu_sc as plsc

VOCAB = 63
VOCAB_PAD = 64
D = 16
SEQ = 19
BATCH = 16384
TOKENS = BATCH * SEQ

NC = 2
NS = 16
NW = NC * NS
BBLK = BATCH // 128
UNITS = SEQ * BBLK
UNITS_W = UNITS // NW
PER_W = UNITS_W * 128


def _pe_np() -> np.ndarray:
    even_i = np.arange(0, D, 2, dtype=np.float32)
    denom = np.power(np.float32(10000.0), even_i / np.float32(D))
    pos = np.arange(SEQ, dtype=np.float32).reshape(SEQ, 1)
    stacked = np.stack([np.sin(pos / denom), np.cos(pos / denom)], axis=-1)
    return stacked.reshape(SEQ, D).astype(np.float32)


_PE = _pe_np()


ROWS = SEQ * VOCAB_PAD


def _sc_body(idx_hbm, tabt_hbm, pe_hbm, out_hbm, idx_v, stage_v, pe_v, tab_v,
             tile_v, isem, ssem0, ssem1, ssem2, ssem3):
    wid = lax.axis_index("s") * NC + lax.axis_index("c")

    idx_cp = pltpu.async_copy(idx_hbm.at[pl.ds(wid * PER_W, PER_W)], idx_v,
                              isem)
    pltpu.sync_copy(tabt_hbm, stage_v.at[pl.ds(0, D * VOCAB)])
    pltpu.sync_copy(pe_hbm, pe_v)

    def _build(d, _):
        for p in range(SEQ):
            pev = plsc.load_gather(pe_v, [jnp.full((16,), p, jnp.int32)
                                          + d * SEQ])
            for c in range(4):
                src = stage_v[pl.ds(d * VOCAB + c * 16, 16)]
                tab_v[d, pl.ds(p * VOCAB_PAD + c * 16, 16)] = src + pev
        return _
    lax.fori_loop(0, D, _build, None)
    idx_cp.wait()

    dvecs = [jnp.full((16,), d, jnp.int32) for d in range(D)]
    sems = (ssem0, ssem1, ssem2, ssem3)
    NB = 4

    def _do_unit(i, k, p, bblk):
        poff = p * VOCAB_PAD
        for g in range(8):
            fidx = idx_v[pl.ds(i * 128 + g * 16, 16)] + poff
            for d in range(D):
                tile_v[k, d, pl.ds(g * 16, 16)] = plsc.load_gather(
                    tab_v, [dvecs[d], fidx])
        pltpu.async_copy(tile_v.at[k, pl.ds(0, 8)], out_hbm.at[p, 0, bblk],
                         sems[k])
        pltpu.async_copy(tile_v.at[k, pl.ds(8, 8)], out_hbm.at[p, 1, bblk],
                         sems[k])

    def _drain(k):
        pltpu.make_async_copy(out_hbm.at[0, 0, 0], tile_v.at[k, pl.ds(0, 8)],
                              sems[k]).wait()
        pltpu.make_async_copy(out_hbm.at[0, 0, 0], tile_v.at[k, pl.ds(8, 8)],
                              sems[k]).wait()

    u0 = wid * UNITS_W
    p0 = lax.div(u0, BBLK)
    b0 = lax.rem(u0, BBLK)

    def _round(j, carry):
        p, bblk = carry
        for k in range(NB):
            @pl.when(j >= 1)
            def _wait():
                _drain(k)
            _do_unit(j * NB + k, k, p, bblk)
            nxt = bblk + 1
            wrap = nxt >= BBLK
            p = lax.select(wrap, p + 1, p)
            bblk = lax.select(wrap, 0, nxt)
        return p, bblk

    lax.fori_loop(0, UNITS_W // NB, _round, (p0, b0))
    for k in range(NB):
        _drain(k)


_sc_call = functools.partial(
    pl.kernel,
    out_type=jax.ShapeDtypeStruct((SEQ, 2, BBLK, 8, 128), jnp.float32),
    mesh=plsc.VectorSubcoreMesh(core_axis_name="c", subcore_axis_name="s"),
    compiler_params=pltpu.CompilerParams(use_tc_tiling_on_sc=False,
                                         needs_layout_passes=False),
    scratch_types=[
        pltpu.VMEM((PER_W,), jnp.int32),
        pltpu.VMEM((D * VOCAB + 16,), jnp.float32),
        pltpu.VMEM((D * SEQ,), jnp.float32),
        pltpu.VMEM((D, SEQ * VOCAB_PAD), jnp.float32),
        pltpu.VMEM((4, D, 128), jnp.float32),
        pltpu.SemaphoreType.DMA,
        pltpu.SemaphoreType.DMA,
        pltpu.SemaphoreType.DMA,
        pltpu.SemaphoreType.DMA,
        pltpu.SemaphoreType.DMA,
    ],
)(_sc_body)


def kernel(batch, embedding_table):
    tabt = embedding_table.astype(jnp.float32).T.reshape(D * VOCAB)
    pe = jnp.asarray(_PE.T.reshape(D * SEQ))
    idx = batch.astype(jnp.int32).T.reshape(TOKENS)
    out5 = _sc_call(idx, tabt, pe)
    out = out5.transpose(2, 4, 0, 1, 3).reshape(BATCH, SEQ, D)
    return out


# speedup vs baseline: 1.3336x; 1.2484x over previous
import functools

import numpy as np
import jax
import jax.numpy as jnp
from jax import lax
from jax.experimental import pallas as pl
from jax.experimental.pallas import tpu as pltpu
from jax.experimental.pallas import tpu_sc as plsc

VOCAB = 63
VOCAB_PAD = 64
D = 16
SEQ = 19
BATCH = 16384
TOKENS = BATCH * SEQ

NC = 2
NS = 16
NW = NC * NS
BBLK = BATCH // 128
UNITS = SEQ * BBLK
UNITS_W = UNITS // NW
PER_W = UNITS_W * 128


def _pe_np() -> np.ndarray:
    even_i = np.arange(0, D, 2, dtype=np.float32)
    denom = np.power(np.float32(10000.0), even_i / np.float32(D))
    pos = np.arange(SEQ, dtype=np.float32).reshape(SEQ, 1)
    stacked = np.stack([np.sin(pos / denom), np.cos(pos / denom)], axis=-1)
    return stacked.reshape(SEQ, D).astype(np.float32)


_PE = _pe_np()


TSTR = 137


def _fuse_body(tab_ref, pe_ref, out_ref):
    out_ref[...] = tab_ref[...][None, :, :] + pe_ref[...][:, None, :]


_fuse = pl.pallas_call(
    _fuse_body,
    out_shape=jax.ShapeDtypeStruct((SEQ, VOCAB_PAD, D), jnp.float32),
)


def _sc_body(idx_hbm, fusedt_hbm, out_hbm, idx_v, tab_v, tile_v, ssem0,
             ssem1):
    wid = lax.axis_index("s") * NC + lax.axis_index("c")

    pltpu.sync_copy(idx_hbm.at[pl.ds(wid * PER_W, PER_W)], idx_v)
    pltpu.sync_copy(fusedt_hbm, tab_v)

    iota = lax.iota(jnp.int32, 16)
    jvecs = [jnp.full((16,), j, jnp.int32) for j in range(16)]
    cvecs = [jnp.full((16,), c, jnp.int32) for c in range(128)]
    sems = (ssem0, ssem1)

    def _do_unit(i, k):
        u = wid * UNITS_W + i
        p = lax.div(u, BBLK)
        bblk = lax.rem(u, BBLK)
        poff = p * VOCAB_PAD
        for g in range(8):
            base = (idx_v[pl.ds(i * 128 + g * 16, 16)] + poff) * D
            for j in range(16):
                gidx = base.at[jvecs[j]].get(mode="promise_in_bounds") + iota
                vals = plsc.load_gather(tab_v, [gidx])
                plsc.store_scatter(tile_v.at[k], [iota, cvecs[g * 16 + j]],
                                   vals)
        pltpu.async_copy(tile_v.at[k, pl.ds(0, 8), pl.ds(0, 128)],
                         out_hbm.at[p, 0, bblk], sems[k])
        pltpu.async_copy(tile_v.at[k, pl.ds(8, 8), pl.ds(0, 128)],
                         out_hbm.at[p, 1, bblk], sems[k])

    def _drain(k):
        pltpu.make_async_copy(out_hbm.at[0, 0, 0],
                              tile_v.at[k, pl.ds(0, 8), pl.ds(0, 128)],
                              sems[k]).wait()
        pltpu.make_async_copy(out_hbm.at[0, 0, 0],
                              tile_v.at[k, pl.ds(8, 8), pl.ds(0, 128)],
                              sems[k]).wait()

    def _pair(j, _):
        for k in range(2):
            @pl.when(j >= 1)
            def _wait():
                _drain(k)
            _do_unit(2 * j + k, k)
        return _

    lax.fori_loop(0, UNITS_W // 2, _pair, None)
    _drain(0)
    _drain(1)


_sc_call = functools.partial(
    pl.kernel,
    out_type=jax.ShapeDtypeStruct((SEQ, 2, BBLK, 8, 128), jnp.float32),
    mesh=plsc.VectorSubcoreMesh(core_axis_name="c", subcore_axis_name="s"),
    compiler_params=pltpu.CompilerParams(use_tc_tiling_on_sc=False,
                                         needs_layout_passes=False),
    scratch_types=[
        pltpu.VMEM((PER_W,), jnp.int32),
        pltpu.VMEM((SEQ * VOCAB_PAD * D,), jnp.float32),
        pltpu.VMEM((2, D, TSTR), jnp.float32),
        pltpu.SemaphoreType.DMA,
        pltpu.SemaphoreType.DMA,
    ],
)(_sc_body)


def kernel(batch, embedding_table):
    tab = jnp.pad(embedding_table.astype(jnp.float32),
                  ((0, VOCAB_PAD - VOCAB), (0, 0)))
    fusedt = _fuse(tab, jnp.asarray(_PE))
    fusedt = fusedt.reshape(SEQ * VOCAB_PAD * D)
    idx = batch.astype(jnp.int32).T.reshape(TOKENS)
    out5 = _sc_call(idx, fusedt)
    out = out5.transpose(2, 4, 0, 1, 3).reshape(BATCH, SEQ, D)
    return out


# speedup vs baseline: 1.3386x; 1.0038x over previous
import functools

import numpy as np
import jax
import jax.numpy as jnp
from jax import lax
from jax.experimental import pallas as pl
from jax.experimental.pallas import tpu as pltpu
from jax.experimental.pallas import tpu_sc as plsc

VOCAB = 63
VOCAB_PAD = 64
D = 16
SEQ = 19
BATCH = 16384
TOKENS = BATCH * SEQ

NC = 2
NS = 16
NW = NC * NS
BBLK = BATCH // 128
UNITS = SEQ * BBLK
UNITS_W = UNITS // NW
PER_W = UNITS_W * 128


def _pe_np() -> np.ndarray:
    even_i = np.arange(0, D, 2, dtype=np.float32)
    denom = np.power(np.float32(10000.0), even_i / np.float32(D))
    pos = np.arange(SEQ, dtype=np.float32).reshape(SEQ, 1)
    stacked = np.stack([np.sin(pos / denom), np.cos(pos / denom)], axis=-1)
    return stacked.reshape(SEQ, D).astype(np.float32)


_PE = _pe_np()


def _fuse_body(tab_ref, pe_ref, out_ref):
    tab_t = jnp.transpose(tab_ref[...], (1, 0))
    pe_t = jnp.transpose(pe_ref[...], (1, 0))
    out_ref[...] = pe_t[:, :, None] + tab_t[:, None, :]


_fuse = pl.pallas_call(
    _fuse_body,
    out_shape=jax.ShapeDtypeStruct((D, SEQ, VOCAB_PAD), jnp.float32),
)


def _sc_body(idx_hbm, fusedt_hbm, out_hbm, idx_v, tab_v, tile_v, ssem0,
             ssem1):
    wid = lax.axis_index("s") * NC + lax.axis_index("c")

    pltpu.sync_copy(idx_hbm.at[pl.ds(wid * PER_W, PER_W)], idx_v)
    pltpu.sync_copy(fusedt_hbm, tab_v)

    dvecs = [jnp.full((16,), d, jnp.int32) for d in range(D)]
    sems = (ssem0, ssem1)

    def _do_unit(i, k):
        u = wid * UNITS_W + i
        p = lax.div(u, BBLK)
        bblk = lax.rem(u, BBLK)
        poff = p * VOCAB_PAD
        for g in range(8):
            fidx = idx_v[pl.ds(i * 128 + g * 16, 16)] + poff
            for d in range(D):
                tile_v[k, d // 8, d % 8, pl.ds(g * 16, 16)] = (
                    plsc.load_gather(tab_v, [dvecs[d], fidx]))
        pltpu.async_copy(tile_v.at[k], out_hbm.at[p, :, bblk], sems[k])

    def _drain(k):
        pltpu.make_async_copy(out_hbm.at[0, :, 0], tile_v.at[k],
                              sems[k]).wait()

    def _pair(j, _):
        for k in range(2):
            @pl.when(j >= 1)
            def _wait():
                _drain(k)
            _do_unit(2 * j + k, k)
        return _

    lax.fori_loop(0, UNITS_W // 2, _pair, None)
    _drain(0)
    _drain(1)


_sc_call = functools.partial(
    pl.kernel,
    out_type=jax.ShapeDtypeStruct((SEQ, 2, BBLK, 8, 128), jnp.float32),
    mesh=plsc.VectorSubcoreMesh(core_axis_name="c", subcore_axis_name="s"),
    compiler_params=pltpu.CompilerParams(use_tc_tiling_on_sc=False,
                                         needs_layout_passes=False),
    scratch_types=[
        pltpu.VMEM((PER_W,), jnp.int32),
        pltpu.VMEM((D, SEQ * VOCAB_PAD), jnp.float32),
        pltpu.VMEM((2, 2, 8, 128), jnp.float32),
        pltpu.SemaphoreType.DMA,
        pltpu.SemaphoreType.DMA,
    ],
)(_sc_body)


def kernel(batch, embedding_table):
    tab = jnp.pad(embedding_table.astype(jnp.float32),
                  ((0, VOCAB_PAD - VOCAB), (0, 0)))
    fusedt = _fuse(tab, jnp.asarray(_PE))
    fusedt = fusedt.reshape(D, SEQ * VOCAB_PAD)
    idx = batch.astype(jnp.int32).T.reshape(TOKENS)
    out5 = _sc_call(idx, fusedt)
    out = out5.transpose(2, 4, 0, 1, 3).reshape(BATCH, SEQ, D)
    return out
